# baseline (device time: 30982 ns/iter reference)
import os

import jax
import jax.numpy as jnp
from jax import lax
from jax.experimental import pallas as pl
from jax.experimental.pallas import tpu as pltpu

_COMM = os.environ.get("KERNEL_COMM", "1") == "1"

N_DEV = 4
N_EXP = 16
E_LOC = 4
CAP = 102
K = 64


def kernel(x, router_W, route_idx, expert_W):
    del router_W
    n, d = x.shape
    _, _, h = expert_W.shape
    c = n // N_DEV
    r = E_LOC * K

    tri = jnp.tril(jnp.ones((c, c), jnp.bfloat16))
    rep = (
        jnp.arange(r, dtype=jnp.int32)[None, :] // K
        == jnp.arange(E_LOC, dtype=jnp.int32)[:, None]
    ).astype(jnp.float32)
    kmod1 = (jnp.arange(r, dtype=jnp.int32)[None, :] % K + 1).astype(
        jnp.float32
    ) * jnp.ones((c, 1), jnp.float32)

    def body(
        x_hbm,
        idx_ref,
        w_hbm,
        tri_ref,
        rep_ref,
        kmod1_ref,
        out_ref,
        x_s,
        w_s,
        keep_buf,
        send_buf,
        recv_buf,
        copy_sems,
        send_sems,
        recv_sems,
    ):
        my_pos = lax.axis_index("i")

        cp_w = [
            pltpu.make_async_copy(w_hbm.at[l], w_s.at[l], copy_sems.at[1 + l])
            for l in range(E_LOC)
        ]
        cp_w[0].start()
        cp_x = pltpu.make_async_copy(x_hbm, x_s, copy_sems.at[0])
        cp_x.start()
        for l in range(1, E_LOC):
            cp_w[l].start()

        if _COMM:
            barrier_sem = pltpu.get_barrier_semaphore()
            for off in (1, 2, 3):
                peer = lax.rem(my_pos + off, N_DEV)
                pl.semaphore_signal(
                    barrier_sem,
                    inc=1,
                    device_id=(peer,),
                    device_id_type=pl.DeviceIdType.MESH,
                )
            pl.semaphore_wait(barrier_sem, N_DEV - 1)

        lt = tri_ref[:, :]

        iota_e = lax.broadcasted_iota(jnp.int32, (c, N_EXP), 1)
        ones16 = jnp.ones((N_EXP, 1), jnp.bfloat16)
        off16 = jnp.zeros((1, N_EXP), jnp.float32)
        for j in range(N_DEV):
            e_blk = idx_ref[j * c : (j + 1) * c, :]
            oh = (e_blk == iota_e).astype(jnp.bfloat16)
            cnt = jnp.dot(lt, oh, preferred_element_type=jnp.float32)
            counts_g = cnt + off16
            off16 = off16 + cnt[c - 1 : c, :]
            ok = (counts_g <= float(CAP)).astype(jnp.bfloat16)
            keep_buf[j * c : (j + 1) * c, :] = jnp.dot(
                ok * oh, ones16, preferred_element_type=jnp.float32
            ).astype(jnp.bfloat16)

        iota_l4 = lax.broadcasted_iota(jnp.int32, (c, E_LOC), 1)

        def gt_cat(j, dev):
            e_blk = idx_ref[pl.ds(j * c, c), :]
            ohd = (e_blk == dev * E_LOC + iota_l4).astype(jnp.bfloat16)
            cnt = jnp.dot(lt, ohd, preferred_element_type=jnp.float32)
            m = (ohd * keep_buf[pl.ds(j * c, c), :]).astype(jnp.float32)
            cnt_exp = jnp.dot(
                cnt * m, rep_ref[:, :], preferred_element_type=jnp.float32
            )
            return (cnt_exp == kmod1_ref[:, :]).astype(jnp.bfloat16)

        gtc = []
        for off in range(N_DEV):
            j = lax.rem(my_pos + off, N_DEV)
            gtc.append(gt_cat(j, my_pos))

        cp_x.wait()
        xg = []
        for off in range(N_DEV):
            j = lax.rem(my_pos + off, N_DEV)
            xj = x_s[pl.ds(j * c, c), :].astype(jnp.bfloat16)
            g = lax.dot_general(
                gtc[off],
                xj,
                (((0,), (0,)), ((), ())),
                preferred_element_type=jnp.float32,
            )
            xg.append(g.astype(jnp.bfloat16))

        sends = []
        for l in range(E_LOC):
            cp_w[l].wait()
            w_bf = w_s[l].astype(jnp.bfloat16)
            stack = jnp.concatenate(
                [xg[off][l * K : (l + 1) * K, :] for off in range(N_DEV)],
                axis=0,
            )
            y = jnp.dot(
                stack, w_bf, preferred_element_type=jnp.float32
            ).astype(jnp.bfloat16)
            recv_buf[my_pos, l * K : (l + 1) * K, :] = y[0:K, :]
            for s, off in enumerate((1, 2, 3)):
                send_buf[s, l * K : (l + 1) * K, :] = y[
                    off * K : (off + 1) * K, :
                ]
                if _COMM:
                    j = lax.rem(my_pos + off, N_DEV)
                    rdma = pltpu.make_async_remote_copy(
                        src_ref=send_buf.at[s, l * K : (l + 1) * K, :],
                        dst_ref=recv_buf.at[my_pos, l * K : (l + 1) * K, :],
                        send_sem=send_sems.at[s, l],
                        recv_sem=recv_sems.at[my_pos, l],
                        device_id=(j,),
                        device_id_type=pl.DeviceIdType.MESH,
                    )
                    rdma.start()
                    sends.append(rdma)

        rgt = [gt_cat(my_pos, lax.rem(my_pos + off, N_DEV)) for off in (1, 2, 3)]

        acc = jnp.dot(
            gtc[0], recv_buf[my_pos], preferred_element_type=jnp.float32
        )
        for i, off in enumerate((1, 2, 3)):
            p = lax.rem(my_pos + off, N_DEV)
            if _COMM:
                for l in range(E_LOC):
                    recv = pltpu.make_async_remote_copy(
                        src_ref=recv_buf.at[p, l * K : (l + 1) * K, :],
                        dst_ref=recv_buf.at[p, l * K : (l + 1) * K, :],
                        send_sem=recv_sems.at[p, l],
                        recv_sem=recv_sems.at[p, l],
                        device_id=(my_pos,),
                        device_id_type=pl.DeviceIdType.MESH,
                    )
                    recv.wait_recv()
            acc = acc + jnp.dot(
                rgt[i], recv_buf[p], preferred_element_type=jnp.float32
            )

        out_ref[:, :] = acc

        for rdma in sends:
            rdma.wait_send()

    return pl.pallas_call(
        body,
        out_shape=jax.ShapeDtypeStruct((c, h), jnp.float32),
        in_specs=[
            pl.BlockSpec(memory_space=pl.ANY),
            pl.BlockSpec(memory_space=pltpu.VMEM),
            pl.BlockSpec(memory_space=pl.ANY),
            pl.BlockSpec(memory_space=pltpu.VMEM),
            pl.BlockSpec(memory_space=pltpu.VMEM),
            pl.BlockSpec(memory_space=pltpu.VMEM),
        ],
        out_specs=pl.BlockSpec(memory_space=pltpu.VMEM),
        scratch_shapes=[
            pltpu.VMEM((n, d), jnp.float32),
            pltpu.VMEM((E_LOC, d, h), jnp.float32),
            pltpu.VMEM((n, 1), jnp.bfloat16),
            pltpu.VMEM((N_DEV - 1, r, h), jnp.bfloat16),
            pltpu.VMEM((N_DEV, r, h), jnp.bfloat16),
            pltpu.SemaphoreType.DMA((1 + E_LOC,)),
            pltpu.SemaphoreType.DMA((N_DEV - 1, E_LOC)),
            pltpu.SemaphoreType.DMA((N_DEV, E_LOC)),
        ],
        compiler_params=(
            pltpu.CompilerParams(collective_id=0) if _COMM else None
        ),
    )(x, route_idx, expert_W, tri, rep, kmod1)


# device time: 29142 ns/iter; 1.0631x vs baseline; 1.0631x over previous
import os

import jax
import jax.numpy as jnp
from jax import lax
from jax.experimental import pallas as pl
from jax.experimental.pallas import tpu as pltpu

_COMM = os.environ.get("KERNEL_COMM", "1") == "1"

N_DEV = 4
N_EXP = 16
E_LOC = 4
CAP = 102
K = 64


def kernel(x, router_W, route_idx, expert_W):
    del router_W
    n, d = x.shape
    _, _, h = expert_W.shape
    c = n // N_DEV
    r = E_LOC * K

    xb = x.astype(jnp.bfloat16)
    wb = expert_W.astype(jnp.bfloat16)

    tri = jnp.tril(jnp.ones((c, c), jnp.bfloat16))
    rep = (
        jnp.arange(r, dtype=jnp.int32)[None, :] // K
        == jnp.arange(E_LOC, dtype=jnp.int32)[:, None]
    ).astype(jnp.float32)
    kmod1 = (jnp.arange(r, dtype=jnp.int32)[None, :] % K + 1).astype(
        jnp.float32
    ) * jnp.ones((c, 1), jnp.float32)

    def body(
        x_hbm,
        idx_ref,
        w_hbm,
        tri_ref,
        rep_ref,
        kmod1_ref,
        out_ref,
        x_s,
        w_s,
        keep_buf,
        send_buf,
        recv_buf,
        copy_sems,
        send_sems,
        recv_sems,
    ):
        my_pos = lax.axis_index("i")

        cp_w = [
            pltpu.make_async_copy(w_hbm.at[l], w_s.at[l], copy_sems.at[1 + l])
            for l in range(E_LOC)
        ]
        cp_w[0].start()
        cp_x = pltpu.make_async_copy(x_hbm, x_s, copy_sems.at[0])
        cp_x.start()
        for l in range(1, E_LOC):
            cp_w[l].start()

        if _COMM:
            barrier_sem = pltpu.get_barrier_semaphore()
            for off in (1, 2, 3):
                peer = lax.rem(my_pos + off, N_DEV)
                pl.semaphore_signal(
                    barrier_sem,
                    inc=1,
                    device_id=(peer,),
                    device_id_type=pl.DeviceIdType.MESH,
                )
            pl.semaphore_wait(barrier_sem, N_DEV - 1)

        lt = tri_ref[:, :]

        iota_e = lax.broadcasted_iota(jnp.int32, (c, N_EXP), 1)
        ones16 = jnp.ones((N_EXP, 1), jnp.bfloat16)
        off16 = jnp.zeros((1, N_EXP), jnp.float32)
        for j in range(N_DEV):
            e_blk = idx_ref[j * c : (j + 1) * c, :]
            oh = (e_blk == iota_e).astype(jnp.bfloat16)
            cnt = jnp.dot(lt, oh, preferred_element_type=jnp.float32)
            counts_g = cnt + off16
            off16 = off16 + cnt[c - 1 : c, :]
            ok = (counts_g <= float(CAP)).astype(jnp.bfloat16)
            keep_buf[j * c : (j + 1) * c, :] = jnp.dot(
                ok * oh, ones16, preferred_element_type=jnp.float32
            ).astype(jnp.bfloat16)

        iota_l4 = lax.broadcasted_iota(jnp.int32, (c, E_LOC), 1)

        def gt_cat(j, dev):
            e_blk = idx_ref[pl.ds(j * c, c), :]
            ohd = (e_blk == dev * E_LOC + iota_l4).astype(jnp.bfloat16)
            cnt = jnp.dot(lt, ohd, preferred_element_type=jnp.float32)
            m = (ohd * keep_buf[pl.ds(j * c, c), :]).astype(jnp.float32)
            cnt_exp = jnp.dot(
                cnt * m, rep_ref[:, :], preferred_element_type=jnp.float32
            )
            return (cnt_exp == kmod1_ref[:, :]).astype(jnp.bfloat16)

        gtc = []
        for off in range(N_DEV):
            j = lax.rem(my_pos + off, N_DEV)
            gtc.append(gt_cat(j, my_pos))

        cp_x.wait()
        xg = []
        for off in range(N_DEV):
            j = lax.rem(my_pos + off, N_DEV)
            xj = x_s[pl.ds(j * c, c), :]
            g = lax.dot_general(
                gtc[off],
                xj,
                (((0,), (0,)), ((), ())),
                preferred_element_type=jnp.float32,
            )
            xg.append(g.astype(jnp.bfloat16))

        sends = []
        for l in range(E_LOC):
            cp_w[l].wait()
            w_bf = w_s[l, :, :]
            stack = jnp.concatenate(
                [xg[off][l * K : (l + 1) * K, :] for off in range(N_DEV)],
                axis=0,
            )
            y = jnp.dot(
                stack, w_bf, preferred_element_type=jnp.float32
            ).astype(jnp.bfloat16)
            recv_buf[my_pos, l * K : (l + 1) * K, :] = y[0:K, :]
            for s, off in enumerate((1, 2, 3)):
                send_buf[s, l * K : (l + 1) * K, :] = y[
                    off * K : (off + 1) * K, :
                ]
                if _COMM:
                    j = lax.rem(my_pos + off, N_DEV)
                    rdma = pltpu.make_async_remote_copy(
                        src_ref=send_buf.at[s, l * K : (l + 1) * K, :],
                        dst_ref=recv_buf.at[my_pos, l * K : (l + 1) * K, :],
                        send_sem=send_sems.at[s, l],
                        recv_sem=recv_sems.at[my_pos, l],
                        device_id=(j,),
                        device_id_type=pl.DeviceIdType.MESH,
                    )
                    rdma.start()
                    sends.append(rdma)

        rgt = [gt_cat(my_pos, lax.rem(my_pos + off, N_DEV)) for off in (1, 2, 3)]

        acc = jnp.dot(
            gtc[0], recv_buf[my_pos], preferred_element_type=jnp.float32
        )
        for i, off in enumerate((1, 2, 3)):
            p = lax.rem(my_pos + off, N_DEV)
            if _COMM:
                for l in range(E_LOC):
                    recv = pltpu.make_async_remote_copy(
                        src_ref=recv_buf.at[p, l * K : (l + 1) * K, :],
                        dst_ref=recv_buf.at[p, l * K : (l + 1) * K, :],
                        send_sem=recv_sems.at[p, l],
                        recv_sem=recv_sems.at[p, l],
                        device_id=(my_pos,),
                        device_id_type=pl.DeviceIdType.MESH,
                    )
                    recv.wait_recv()
            acc = acc + jnp.dot(
                rgt[i], recv_buf[p], preferred_element_type=jnp.float32
            )

        out_ref[:, :] = acc

        for rdma in sends:
            rdma.wait_send()

    return pl.pallas_call(
        body,
        out_shape=jax.ShapeDtypeStruct((c, h), jnp.float32),
        in_specs=[
            pl.BlockSpec(memory_space=pl.ANY),
            pl.BlockSpec(memory_space=pltpu.VMEM),
            pl.BlockSpec(memory_space=pl.ANY),
            pl.BlockSpec(memory_space=pltpu.VMEM),
            pl.BlockSpec(memory_space=pltpu.VMEM),
            pl.BlockSpec(memory_space=pltpu.VMEM),
        ],
        out_specs=pl.BlockSpec(memory_space=pltpu.VMEM),
        scratch_shapes=[
            pltpu.VMEM((n, d), jnp.bfloat16),
            pltpu.VMEM((E_LOC, d, h), jnp.bfloat16),
            pltpu.VMEM((n, 1), jnp.bfloat16),
            pltpu.VMEM((N_DEV - 1, r, h), jnp.bfloat16),
            pltpu.VMEM((N_DEV, r, h), jnp.bfloat16),
            pltpu.SemaphoreType.DMA((1 + E_LOC,)),
            pltpu.SemaphoreType.DMA((N_DEV - 1, E_LOC)),
            pltpu.SemaphoreType.DMA((N_DEV, E_LOC)),
        ],
        compiler_params=(
            pltpu.CompilerParams(collective_id=0) if _COMM else None
        ),
    )(xb, route_idx, wb, tri, rep, kmod1)
